# baseline (speedup 1.0000x reference)
import functools

import jax
import jax.numpy as jnp
from jax import lax
from jax.experimental import pallas as pl
from jax.experimental.pallas import tpu as pltpu
from jax.experimental.pallas import tpu_sc as plsc

VOCAB = 1000000
EMBED_DIM = 128
BATCH = 4096
HIST_LEN = 200

NC = 2
NS = 16
NW = NC * NS

B = BATCH * HIST_LEN
ROWS_PER_W = B // NW
CHUNK = 128
NCH = ROWS_PER_W // CHUNK
NBUF = 6
LEAD = NBUF // 2
NCH_MAIN = NCH - (NCH % NBUF)

_mesh = plsc.VectorSubcoreMesh(core_axis_name="c", subcore_axis_name="s")


@functools.partial(
    pl.kernel,
    out_type=jax.ShapeDtypeStruct((B, EMBED_DIM), jnp.float32),
    mesh=_mesh,
    scratch_types=(
        [pltpu.VMEM((NCH, CHUNK), jnp.int32)]
        + [pltpu.VMEM((CHUNK, EMBED_DIM), jnp.float32)] * NBUF
        + [pltpu.SemaphoreType.DMA] * (2 * NBUF)
    ),
)
def _gather_kernel(table_hbm, idx_hbm, out_hbm, idx_v, *bufs):
    rows = list(bufs[:NBUF])
    gs = list(bufs[NBUF:2 * NBUF])
    os_ = list(bufs[2 * NBUF:])

    wid = lax.axis_index("s") * NC + lax.axis_index("c")
    base = wid * ROWS_PER_W

    def start_gather(c, b):
        pltpu.async_copy(table_hbm.at[idx_v.at[c]], rows[b], gs[b])

    def wait_gather(c, b):
        pltpu.make_async_copy(table_hbm.at[idx_v.at[c]], rows[b], gs[b]).wait()

    def start_out(c, b):
        pltpu.async_copy(rows[b], out_hbm.at[pl.ds(base + c * CHUNK, CHUNK)],
                         os_[b])

    def wait_out(c, b):
        pltpu.make_async_copy(
            rows[b], out_hbm.at[pl.ds(base + c * CHUNK, CHUNK)], os_[b]).wait()

    pltpu.sync_copy(idx_hbm.at[wid], idx_v)

    for c in range(LEAD):
        start_gather(c, c)
    for s in range(NBUF):
        if s + LEAD < NBUF:
            start_gather(s + LEAD, s + LEAD)
        else:
            wait_out(s - LEAD, (s + LEAD) % NBUF)
            start_gather(s + LEAD, (s + LEAD) % NBUF)
        wait_gather(s, s)
        start_out(s, s)

    @pl.loop(NBUF, NCH_MAIN, step=NBUF)
    def _(g):
        for j in range(NBUF):
            s = g + j
            bn = (j + LEAD) % NBUF

            @pl.when(s + LEAD < NCH)
            def _():
                wait_out(s - LEAD, bn)
                start_gather(s + LEAD, bn)

            wait_gather(s, j)
            start_out(s, j)

    for s in range(NCH_MAIN, NCH):
        wait_gather(s, s % NBUF)
        start_out(s, s % NBUF)

    for c in range(NCH - 2 * LEAD, NCH):
        wait_out(c, c % NBUF)


def kernel(token_seq, table):
    idx = token_seq.reshape(NW, NCH, CHUNK)
    out = _gather_kernel(table, idx)
    return out.reshape(BATCH, HIST_LEN, EMBED_DIM)


# speedup vs baseline: 1.0016x; 1.0016x over previous
import functools

import jax
import jax.numpy as jnp
from jax import lax
from jax.experimental import pallas as pl
from jax.experimental.pallas import tpu as pltpu
from jax.experimental.pallas import tpu_sc as plsc

VOCAB = 1000000
EMBED_DIM = 128
BATCH = 4096
HIST_LEN = 200

NC = 2
NS = 16
NW = NC * NS

B = BATCH * HIST_LEN
ROWS_PER_W = B // NW
CHUNK = 128
NCH = ROWS_PER_W // CHUNK
PAIR = 2 * CHUNK
NPAIR = NCH // 2
NBUF = 3

_mesh = plsc.VectorSubcoreMesh(core_axis_name="c", subcore_axis_name="s")


@functools.partial(
    pl.kernel,
    out_type=jax.ShapeDtypeStruct((B, EMBED_DIM), jnp.float32),
    mesh=_mesh,
    scratch_types=(
        [pltpu.VMEM((NCH, CHUNK), jnp.int32)]
        + [pltpu.VMEM((PAIR, EMBED_DIM), jnp.float32)] * NBUF
        + [pltpu.SemaphoreType.DMA] * (2 * NBUF)
    ),
)
def _gather_kernel(table_hbm, idx_hbm, out_hbm, idx_v, *bufs):
    rows = list(bufs[:NBUF])
    gs = list(bufs[NBUF:2 * NBUF])
    os_ = list(bufs[2 * NBUF:])

    wid = lax.axis_index("s") * NC + lax.axis_index("c")
    base = wid * ROWS_PER_W

    def start_gathers(p, b):
        pltpu.async_copy(table_hbm.at[idx_v.at[2 * p]],
                         rows[b].at[pl.ds(0, CHUNK)], gs[b])
        pltpu.async_copy(table_hbm.at[idx_v.at[2 * p + 1]],
                         rows[b].at[pl.ds(CHUNK, CHUNK)], gs[b])

    def wait_gathers(p, b):
        pltpu.make_async_copy(table_hbm.at[idx_v.at[2 * p]],
                              rows[b].at[pl.ds(0, CHUNK)], gs[b]).wait()
        pltpu.make_async_copy(table_hbm.at[idx_v.at[2 * p + 1]],
                              rows[b].at[pl.ds(CHUNK, CHUNK)], gs[b]).wait()

    def start_out(p, b):
        pltpu.async_copy(rows[b], out_hbm.at[pl.ds(base + p * PAIR, PAIR)],
                         os_[b])

    def wait_out(p, b):
        pltpu.make_async_copy(
            rows[b], out_hbm.at[pl.ds(base + p * PAIR, PAIR)], os_[b]).wait()

    pltpu.sync_copy(idx_hbm.at[wid], idx_v)

    start_gathers(0, 0)
    start_gathers(1, 1)
    wait_gathers(0, 0)
    start_out(0, 0)
    start_gathers(2, 2)
    wait_gathers(1, 1)
    start_out(1, 1)

    @pl.loop(NBUF, NPAIR - 1, step=NBUF)
    def _(g):
        for j in range(NBUF):
            p = g + j
            wait_out(p - NBUF, j)
            start_gathers(p, j)
            wait_gathers(p - 1, (j + 2) % NBUF)
            start_out(p - 1, (j + 2) % NBUF)

    wait_out(NPAIR - 1 - NBUF, (NPAIR - 1) % NBUF)
    start_gathers(NPAIR - 1, (NPAIR - 1) % NBUF)
    wait_gathers(NPAIR - 2, (NPAIR - 2) % NBUF)
    start_out(NPAIR - 2, (NPAIR - 2) % NBUF)
    wait_gathers(NPAIR - 1, (NPAIR - 1) % NBUF)
    start_out(NPAIR - 1, (NPAIR - 1) % NBUF)
    for p in range(NPAIR - NBUF, NPAIR):
        wait_out(p, p % NBUF)


def kernel(token_seq, table):
    idx = token_seq.reshape(NW, NCH, CHUNK)
    out = _gather_kernel(table, idx)
    return out.reshape(BATCH, HIST_LEN, EMBED_DIM)
